# Initial kernel scaffold; baseline (speedup 1.0000x reference)
import jax
import jax.numpy as jnp
from jax.experimental import pallas as pl


def kernel(index, targets, table):
    raise NotImplementedError("write your pallas kernel here")


# speedup vs baseline: 1.2135x; 1.2135x over previous
import functools

import jax
import jax.numpy as jnp
from jax import lax
from jax.experimental import pallas as pl
from jax.experimental.pallas import tpu as pltpu
from jax.experimental.pallas import tpu_sc as plsc

B, T, V = 1024, 200, 1000
N = B * T
NW = 32
ROWS_PER_W = N // NW
CHUNK = 32
NCHUNK = ROWS_PER_W // CHUNK


def _row_lse_body(table_ref, lse_ref):
    x = table_ref[...]
    m = jnp.max(x, axis=1)
    s = jnp.sum(jnp.exp(x - m[:, None]), axis=1)
    lse_ref[...] = jnp.log(s) + m


def _row_lse(table):
    return pl.pallas_call(
        _row_lse_body,
        out_shape=jax.ShapeDtypeStruct((V,), jnp.float32),
    )(table)


_sc_mesh = plsc.VectorSubcoreMesh(core_axis_name="c", subcore_axis_name="s")


@functools.partial(
    pl.kernel,
    mesh=_sc_mesh,
    out_type=(
        jax.ShapeDtypeStruct((N, V), jnp.float32),
        jax.ShapeDtypeStruct((NW * 16,), jnp.float32),
    ),
    scratch_types=[
        pltpu.VMEM((CHUNK,), jnp.int32),
        pltpu.VMEM((CHUNK,), jnp.int32),
        pltpu.VMEM((CHUNK,), jnp.int32),
        pltpu.VMEM((CHUNK,), jnp.float32),
        pltpu.VMEM((CHUNK,), jnp.float32),
        pltpu.VMEM((CHUNK, V), jnp.float32),
        pltpu.VMEM((16,), jnp.float32),
        pltpu.SemaphoreType.DMA,
        pltpu.SemaphoreType.DMA,
        pltpu.SemaphoreType.DMA,
    ],
    compiler_params=pltpu.CompilerParams(use_tc_tiling_on_sc=False),
)
def _sc_gather(idx_hbm, tgt_hbm, lse_hbm, table_hbm, tableflat_hbm,
               out_hbm, part_hbm,
               idx_v, tgt_v, flat_v, picked_v, lsev_v, rows_v, part_v,
               sem, sem2, sem3):
    wid = lax.axis_index("s") * 2 + lax.axis_index("c")
    base = wid * ROWS_PER_W

    def body(i, acc):
        r0 = base + i * CHUNK
        pltpu.sync_copy(idx_hbm.at[pl.ds(r0, CHUNK)], idx_v)
        pltpu.sync_copy(tgt_hbm.at[pl.ds(r0, CHUNK)], tgt_v)
        for j in range(CHUNK // 16):
            tgt16 = tgt_v[pl.ds(j * 16, 16)]
            idx16 = idx_v[pl.ds(j * 16, 16)]
            flat_v[pl.ds(j * 16, 16)] = idx16 * V + tgt16
        copy_rows = pltpu.async_copy(table_hbm.at[idx_v], rows_v, sem)
        copy_picked = pltpu.async_copy(
            tableflat_hbm.at[flat_v], picked_v, sem2)
        copy_lse = pltpu.async_copy(lse_hbm.at[idx_v], lsev_v, sem3)
        copy_picked.wait()
        copy_lse.wait()
        for j in range(CHUNK // 16):
            acc = acc + (lsev_v[pl.ds(j * 16, 16)]
                         - picked_v[pl.ds(j * 16, 16)])
        copy_rows.wait()
        pltpu.sync_copy(rows_v, out_hbm.at[pl.ds(r0, CHUNK)])
        return acc

    acc = lax.fori_loop(0, NCHUNK, body, jnp.zeros((16,), jnp.float32))
    part_v[...] = acc
    pltpu.sync_copy(part_v, part_hbm.at[pl.ds(wid * 16, 16)])


def kernel(index, targets, table):
    idx_flat = index.reshape(N)
    tgt_flat = targets.reshape(N)
    lse = _row_lse(table)
    tableflat = lax.optimization_barrier(table).reshape(V * V)
    logits, partials = _sc_gather(idx_flat, tgt_flat, lse, table,
                                  tableflat)
    loss = jnp.sum(partials) * (1.0 / N)
    return (logits, loss)


# speedup vs baseline: 1.2951x; 1.0672x over previous
import functools

import jax
import jax.numpy as jnp
from jax import lax
from jax.experimental import pallas as pl
from jax.experimental.pallas import tpu as pltpu
from jax.experimental.pallas import tpu_sc as plsc

B, T, V = 1024, 200, 1000
N = B * T
NW = 32
ROWS_PER_W = N // NW
CHUNK = 32
NCHUNK = ROWS_PER_W // CHUNK


def _row_lse_body(table_ref, lse_ref):
    x = table_ref[...]
    m = jnp.max(x, axis=1)
    s = jnp.sum(jnp.exp(x - m[:, None]), axis=1)
    lse_ref[...] = jnp.log(s) + m


def _row_lse(table):
    return pl.pallas_call(
        _row_lse_body,
        out_shape=jax.ShapeDtypeStruct((V,), jnp.float32),
    )(table)


_sc_mesh = plsc.VectorSubcoreMesh(core_axis_name="c", subcore_axis_name="s")


@functools.partial(
    pl.kernel,
    mesh=_sc_mesh,
    out_type=(
        jax.ShapeDtypeStruct((N, V), jnp.float32),
        jax.ShapeDtypeStruct((NW * 16,), jnp.float32),
    ),
    scratch_types=[
        pltpu.VMEM((ROWS_PER_W,), jnp.int32),
        pltpu.VMEM((ROWS_PER_W,), jnp.int32),
        [pltpu.VMEM((CHUNK,), jnp.int32)] * 2,
        [pltpu.VMEM((CHUNK,), jnp.int32)] * 2,
        [pltpu.VMEM((CHUNK,), jnp.float32)] * 2,
        [pltpu.VMEM((CHUNK,), jnp.float32)] * 2,
        [pltpu.VMEM((CHUNK, V), jnp.float32)] * 2,
        pltpu.VMEM((16,), jnp.float32),
        [pltpu.SemaphoreType.DMA] * 2,
        [pltpu.SemaphoreType.DMA] * 2,
        [pltpu.SemaphoreType.DMA] * 2,
        [pltpu.SemaphoreType.DMA] * 2,
    ],
    compiler_params=pltpu.CompilerParams(use_tc_tiling_on_sc=False),
)
def _sc_gather(idx_hbm, tgt_hbm, lse_hbm, table_hbm, tableflat_hbm,
               out_hbm, part_hbm,
               idx_all, tgt_all, idxb, flat, pk, ls, rows, part_v,
               sg, swb, spk, sls):
    wid = lax.axis_index("s") * 2 + lax.axis_index("c")
    base = wid * ROWS_PER_W
    pltpu.sync_copy(idx_hbm.at[pl.ds(base, ROWS_PER_W)], idx_all)
    pltpu.sync_copy(tgt_hbm.at[pl.ds(base, ROWS_PER_W)], tgt_all)

    def issue(c, p):
        off = c * CHUNK
        for j in range(CHUNK // 16):
            idx16 = idx_all[pl.ds(off + j * 16, 16)]
            tgt16 = tgt_all[pl.ds(off + j * 16, 16)]
            idxb[p][pl.ds(j * 16, 16)] = idx16
            flat[p][pl.ds(j * 16, 16)] = idx16 * V + tgt16
        pltpu.async_copy(tableflat_hbm.at[flat[p]], pk[p], spk[p])
        pltpu.async_copy(lse_hbm.at[idxb[p]], ls[p], sls[p])
        pltpu.async_copy(table_hbm.at[idxb[p]], rows[p], sg[p])

    def wait_rows(p):
        pltpu.make_async_copy(table_hbm.at[idxb[p]], rows[p], sg[p]).wait()

    def start_wb(c, p):
        pltpu.async_copy(
            rows[p], out_hbm.at[pl.ds(base + c * CHUNK, CHUNK)], swb[p])

    def wait_wb(p):
        pltpu.make_async_copy(
            rows[p], out_hbm.at[pl.ds(base, CHUNK)], swb[p]).wait()

    def acc_chunk(p, acc):
        pltpu.make_async_copy(tableflat_hbm.at[flat[p]], pk[p],
                              spk[p]).wait()
        pltpu.make_async_copy(lse_hbm.at[idxb[p]], ls[p], sls[p]).wait()
        for j in range(CHUNK // 16):
            acc = acc + (ls[p][pl.ds(j * 16, 16)]
                         - pk[p][pl.ds(j * 16, 16)])
        return acc

    acc0 = jnp.zeros((16,), jnp.float32)
    issue(0, 0)
    issue(1, 1)
    wait_rows(0)
    start_wb(0, 0)
    acc0 = acc_chunk(0, acc0)

    def body(g, acc):
        c0 = 2 * g
        wait_wb(0)
        issue(c0, 0)
        wait_rows(1)
        start_wb(c0 - 1, 1)
        acc = acc_chunk(1, acc)
        wait_wb(1)
        issue(c0 + 1, 1)
        wait_rows(0)
        start_wb(c0, 0)
        acc = acc_chunk(0, acc)
        return acc

    acc0 = lax.fori_loop(1, NCHUNK // 2, body, acc0)
    wait_rows(1)
    start_wb(NCHUNK - 1, 1)
    acc0 = acc_chunk(1, acc0)
    wait_wb(0)
    wait_wb(1)
    part_v[...] = acc0
    pltpu.sync_copy(part_v, part_hbm.at[pl.ds(wid * 16, 16)])


def kernel(index, targets, table):
    idx_flat = index.reshape(N)
    tgt_flat = targets.reshape(N)
    lse = _row_lse(table)
    tableflat = lax.optimization_barrier(table).reshape(V * V)
    logits, partials = _sc_gather(idx_flat, tgt_flat, lse, table,
                                  tableflat)
    loss = jnp.sum(partials) * (1.0 / N)
    return (logits, loss)


# speedup vs baseline: 1.8816x; 1.4528x over previous
import functools

import jax
import jax.numpy as jnp
from jax import lax
from jax.experimental import pallas as pl
from jax.experimental.pallas import tpu as pltpu
from jax.experimental.pallas import tpu_sc as plsc

B, T, V = 1024, 200, 1000
VP = 1024
NT = VP // 128
N = B * T
NW = 32
ROWS_PER_W = N // NW
CHUNK = 32
NCHUNK = ROWS_PER_W // CHUNK


def _row_lse_body(table_ref, lse_ref):
    x = table_ref[...]
    m = jnp.max(x, axis=1)
    s = jnp.sum(jnp.exp(x - m[:, None]), axis=1)
    lse_ref[...] = jnp.log(s) + m


def _row_lse(table):
    return pl.pallas_call(
        _row_lse_body,
        out_shape=jax.ShapeDtypeStruct((V,), jnp.float32),
    )(table)


_sc_mesh = plsc.VectorSubcoreMesh(core_axis_name="c", subcore_axis_name="s")


@functools.partial(
    pl.kernel,
    mesh=_sc_mesh,
    out_type=(
        jax.ShapeDtypeStruct((N, V), jnp.float32),
        jax.ShapeDtypeStruct((NW, 16), jnp.float32),
    ),
    scratch_types=[
        pltpu.VMEM((ROWS_PER_W,), jnp.int32),
        pltpu.VMEM((ROWS_PER_W,), jnp.int32),
        [pltpu.VMEM((CHUNK,), jnp.int32)] * 2,
        [pltpu.VMEM((CHUNK,), jnp.int32)] * 2,
        [pltpu.VMEM((CHUNK,), jnp.float32)] * 2,
        [pltpu.VMEM((CHUNK,), jnp.float32)] * 2,
        [pltpu.VMEM((CHUNK, V), jnp.float32)] * 2,
        [pltpu.VMEM((CHUNK, 128), jnp.float32)] * 2,
        pltpu.VMEM((16,), jnp.float32),
        [pltpu.SemaphoreType.DMA] * 2,
        [pltpu.SemaphoreType.DMA] * 2,
        [pltpu.SemaphoreType.DMA] * 2,
        [pltpu.SemaphoreType.DMA] * 2,
    ],
    compiler_params=pltpu.CompilerParams(use_tc_tiling_on_sc=True),
)
def _sc_gather(idx_hbm, tgt_hbm, lse_hbm, piece_hbm,
               tableflat_hbm, out_hbm, part_hbm,
               idx_all, tgt_all, idxb, flat, pk, ls, rows, tail,
               part_v, sg, swb, spk, sls):
    wid = lax.axis_index("s") * 2 + lax.axis_index("c")
    base = wid * ROWS_PER_W
    pltpu.sync_copy(idx_hbm.at[pl.ds(base, ROWS_PER_W)], idx_all)
    pltpu.sync_copy(tgt_hbm.at[pl.ds(base, ROWS_PER_W)], tgt_all)
    lanes = lax.iota(jnp.int32, 16)

    def issue(c, p):
        off = c * CHUNK
        for j in range(CHUNK // 16):
            idx16 = idx_all[pl.ds(off + j * 16, 16)]
            tgt16 = tgt_all[pl.ds(off + j * 16, 16)]
            idxb[p][pl.ds(j * 16, 16)] = idx16
            flat[p][pl.ds(j * 16, 16)] = idx16 * V + tgt16
        pltpu.async_copy(tableflat_hbm.at[flat[p]], pk[p], spk[p])
        pltpu.async_copy(lse_hbm.at[idxb[p]], ls[p], sls[p])
        for h in range(CHUNK // 16):
            idx16 = idx_all[pl.ds(off + h * 16, 16)]
            p8 = idx16 * NT
            for tc in range(NT - 1):
                pltpu.async_copy(
                    piece_hbm.at[p8 + tc],
                    rows[p].at[pl.ds(h * 16, 16), pl.ds(tc * 128, 128)],
                    sg[p])
            pltpu.async_copy(piece_hbm.at[p8 + (NT - 1)],
                             tail[p].at[pl.ds(h * 16, 16)],
                             sg[p])

    def wait_rows(p):
        for _ in range(2 * NT):
            pltpu.make_async_copy(
                piece_hbm.at[lanes],
                rows[p].at[pl.ds(0, 16), pl.ds(0, 128)],
                sg[p]).wait()

    def fix_tail(p):
        for r in range(CHUNK):
            rows[p][r, pl.ds(984, 16)] = tail[p][r, pl.ds(88, 16)]
            for j in range(6):
                rows[p][r, pl.ds(896 + 16 * j, 16)] = (
                    tail[p][r, pl.ds(16 * j, 16)])

    def start_wb(c, p):
        pltpu.async_copy(rows[p],
                         out_hbm.at[pl.ds(base + c * CHUNK, CHUNK)],
                         swb[p])

    def wait_wb(p):
        pltpu.make_async_copy(rows[p],
                              out_hbm.at[pl.ds(base, CHUNK)],
                              swb[p]).wait()

    def acc_chunk(p, acc):
        pltpu.make_async_copy(tableflat_hbm.at[flat[p]], pk[p],
                              spk[p]).wait()
        pltpu.make_async_copy(lse_hbm.at[idxb[p]], ls[p], sls[p]).wait()
        for j in range(CHUNK // 16):
            acc = acc + (ls[p][pl.ds(j * 16, 16)]
                         - pk[p][pl.ds(j * 16, 16)])
        return acc

    acc0 = jnp.zeros((16,), jnp.float32)
    issue(0, 0)
    issue(1, 1)
    wait_rows(0)
    fix_tail(0)
    start_wb(0, 0)
    acc0 = acc_chunk(0, acc0)

    def body(g, acc):
        c0 = 2 * g
        wait_wb(0)
        issue(c0, 0)
        wait_rows(1)
        fix_tail(1)
        start_wb(c0 - 1, 1)
        acc = acc_chunk(1, acc)
        wait_wb(1)
        issue(c0 + 1, 1)
        wait_rows(0)
        fix_tail(0)
        start_wb(c0, 0)
        acc = acc_chunk(0, acc)
        return acc

    acc0 = lax.fori_loop(1, NCHUNK // 2, body, acc0)
    wait_rows(1)
    fix_tail(1)
    start_wb(NCHUNK - 1, 1)
    acc0 = acc_chunk(1, acc0)
    wait_wb(0)
    wait_wb(1)
    part_v[...] = acc0
    pltpu.sync_copy(part_v, part_hbm.at[wid])


def kernel(index, targets, table):
    idx_flat = index.reshape(N)
    tgt_flat = targets.reshape(N)
    lse = _row_lse(table)
    pieces = jnp.pad(table, ((0, 0), (0, VP - V))).reshape(V * NT, 128)
    tableflat = lax.optimization_barrier(table).reshape(V * V)
    logits, partials = _sc_gather(idx_flat, tgt_flat, lse, pieces,
                                  tableflat)
    loss = jnp.sum(partials) * (1.0 / N)
    return (logits, loss)
